# Initial kernel scaffold; baseline (speedup 1.0000x reference)
import functools

import jax
import jax.numpy as jnp
from jax.experimental import pallas as pl
from jax.experimental.pallas import tpu as pltpu

_TM = 512


def _routing(x_blk, gate_wt):
    e_pad = gate_wt.shape[1]
    logits = jax.lax.dot_general(
        x_blk, gate_wt, (((1,), (0,)), ((), ())),
        preferred_element_type=jnp.float32,
    )
    lane = jax.lax.broadcasted_iota(jnp.int32, logits.shape, 1)
    neg = jnp.float32(-jnp.inf)
    valid = lane < 8
    logits = jnp.where(valid, logits, neg)
    m1 = jnp.max(logits, axis=1)
    is1 = logits == m1[:, None]
    i1 = jnp.min(jnp.where(is1, lane, e_pad), axis=1)
    l2 = jnp.where(lane == i1[:, None], neg, logits)
    m2 = jnp.max(l2, axis=1)
    is2 = l2 == m2[:, None]
    i2 = jnp.min(jnp.where(is2, lane, e_pad), axis=1)
    w1 = 1.0 / (1.0 + jnp.exp(m2 - m1))
    w2 = 1.0 - w1
    return i1, i2, w1, w2


def _moe_dense_body(x_ref, gate_wt_ref, wt_ref, b_ref, out_ref):
    e = pl.program_id(1)
    x = x_ref[...]
    i1, i2, w1, w2 = _routing(x, gate_wt_ref[...])
    w_e = (jnp.where(i1 == e, w1, 0.0) + jnp.where(i2 == e, w2, 0.0))
    xb = x.astype(jnp.bfloat16)
    y = jax.lax.dot_general(
        xb, wt_ref[0], (((1,), (0,)), ((), ())),
        preferred_element_type=jnp.float32,
    )
    y = y + b_ref[...]
    contrib = y * w_e[:, None]

    @pl.when(e == 0)
    def _init():
        out_ref[...] = contrib

    @pl.when(e != 0)
    def _acc():
        out_ref[...] += contrib


@functools.partial(jax.jit, static_argnames=())
def kernel(inputs, gate_w, expert_w, expert_b):
    b, s, d_in = inputs.shape
    e, d_out, _ = expert_w.shape
    t = b * s
    x2 = inputs.reshape(t, d_in)
    gate_wt = jnp.zeros((d_in, 128), jnp.float32).at[:, :e].set(gate_w.T)
    wt = jnp.swapaxes(expert_w, 1, 2).astype(jnp.bfloat16)

    grid = (t // _TM, e)
    out = pl.pallas_call(
        _moe_dense_body,
        grid=grid,
        in_specs=[
            pl.BlockSpec((_TM, d_in), lambda i, j: (i, 0)),
            pl.BlockSpec((d_in, 128), lambda i, j: (0, 0)),
            pl.BlockSpec((1, d_in, d_out), lambda i, j: (j, 0, 0)),
            pl.BlockSpec((1, d_out), lambda i, j: (j, 0)),
        ],
        out_specs=pl.BlockSpec((_TM, d_out), lambda i, j: (i, 0)),
        out_shape=jax.ShapeDtypeStruct((t, d_out), jnp.float32),
        compiler_params=pltpu.CompilerParams(
            dimension_semantics=("parallel", "arbitrary"),
        ),
    )(x2, gate_wt, wt, expert_b)
    return out.reshape(b, s, d_out)


# speedup vs baseline: 1.0278x; 1.0278x over previous
import functools

import jax
import jax.numpy as jnp
from jax.experimental import pallas as pl
from jax.experimental.pallas import tpu as pltpu

_TM = 512


def _routing(x_blk, gate_wt, n_e):
    e_pad = gate_wt.shape[1]
    logits = jax.lax.dot_general(
        x_blk, gate_wt, (((1,), (0,)), ((), ())),
        preferred_element_type=jnp.float32,
    )
    lane = jax.lax.broadcasted_iota(jnp.int32, logits.shape, 1)
    neg = jnp.float32(-jnp.inf)
    valid = lane < n_e
    logits = jnp.where(valid, logits, neg)
    m1 = jnp.max(logits, axis=1)
    is1 = logits == m1[:, None]
    i1 = jnp.min(jnp.where(is1, lane, e_pad), axis=1)
    l2 = jnp.where(lane == i1[:, None], neg, logits)
    m2 = jnp.max(l2, axis=1)
    is2 = l2 == m2[:, None]
    i2 = jnp.min(jnp.where(is2, lane, e_pad), axis=1)
    w1 = 1.0 / (1.0 + jnp.exp(m2 - m1))
    w2 = 1.0 - w1
    return i1, i2, w1, w2


def _moe_dense_body(x_ref, gate_wt_ref, wt_ref, b_ref, out_ref):
    n_e = pl.num_programs(1)
    e = pl.program_id(1)
    x = x_ref[...]
    i1, i2, w1, w2 = _routing(x, gate_wt_ref[...], n_e)
    w_e = (jnp.where(i1 == e, w1, 0.0) + jnp.where(i2 == e, w2, 0.0))
    xb = x.astype(jnp.bfloat16)
    y = jax.lax.dot_general(
        xb, wt_ref[0], (((1,), (0,)), ((), ())),
        preferred_element_type=jnp.float32,
    )
    y = y + b_ref[0]
    contrib = y * w_e[:, None]

    @pl.when(e == 0)
    def _init():
        out_ref[...] = contrib

    @pl.when(e != 0)
    def _acc():
        out_ref[...] += contrib


@functools.partial(jax.jit, static_argnames=())
def kernel(inputs, gate_w, expert_w, expert_b):
    b, s, d_in = inputs.shape
    e, d_out, _ = expert_w.shape
    t = b * s
    x2 = inputs.reshape(t, d_in)
    gate_wt = jnp.zeros((d_in, 128), jnp.float32).at[:, :e].set(gate_w.T)
    wt = jnp.swapaxes(expert_w, 1, 2).astype(jnp.bfloat16)

    grid = (t // _TM, e)
    out = pl.pallas_call(
        _moe_dense_body,
        grid=grid,
        in_specs=[
            pl.BlockSpec((_TM, d_in), lambda i, j: (i, 0)),
            pl.BlockSpec((d_in, 128), lambda i, j: (0, 0)),
            pl.BlockSpec((1, d_in, d_out), lambda i, j: (j, 0, 0)),
            pl.BlockSpec((1, 1, d_out), lambda i, j: (j, 0, 0)),
        ],
        out_specs=pl.BlockSpec((_TM, d_out), lambda i, j: (i, 0)),
        out_shape=jax.ShapeDtypeStruct((t, d_out), jnp.float32),
        compiler_params=pltpu.CompilerParams(
            dimension_semantics=("parallel", "arbitrary"),
        ),
    )(x2, gate_wt, wt, expert_b.reshape(e, 1, d_out))
    return out.reshape(b, s, d_out)
